# Initial kernel scaffold; baseline (speedup 1.0000x reference)
import jax
import jax.numpy as jnp
from jax.experimental import pallas as pl


def kernel(user_emb_weight, item_emb_weight, edge_index, edge_weight):
    raise NotImplementedError("write your pallas kernel here")


# speedup vs baseline: 3.1881x; 3.1881x over previous
import functools

import jax
import jax.numpy as jnp
from jax import lax
from jax.experimental import pallas as pl
from jax.experimental.pallas import tpu as pltpu
from jax.experimental.pallas import tpu_sc as plsc

_NC = 2
_NS = 16
_NW = _NC * _NS
_C = 128


def _largest_divisor_leq(n, cap):
    for d in range(min(cap, n), 0, -1):
        if n % d == 0:
            return d
    return 1


@functools.lru_cache(maxsize=None)
def _make_layer(Np, D, ncw):
    rpt = Np // _NS
    cz = _largest_divisor_leq(rpt, 128)
    nz = rpt // cz
    mesh = plsc.VectorSubcoreMesh(
        core_axis_name="c", subcore_axis_name="s",
        num_cores=_NC, num_subcores=_NS)

    @functools.partial(
        pl.kernel,
        out_type=jax.ShapeDtypeStruct((_NC, Np, D), jnp.float32),
        mesh=mesh,
        scratch_types=[
            pltpu.VMEM((ncw, _C), jnp.int32),
            pltpu.VMEM((ncw, _C), jnp.int32),
            pltpu.VMEM((_C * 16,), jnp.float32),
            pltpu.VMEM((_C, D), jnp.float32),
            pltpu.VMEM_SHARED((Np, D), jnp.float32),
            pltpu.SemaphoreType.DMA,
        ],
    )
    def layer(emb_hbm, col_hbm, row_hbm, w_hbm, part_hbm,
              colv, rowv, wv, rows_v, acc, sem):
        cid = lax.axis_index("c")
        sid = lax.axis_index("s")
        wid = sid * _NC + cid

        pltpu.sync_copy(col_hbm.at[wid], colv)
        pltpu.sync_copy(row_hbm.at[wid], rowv)

        zero16 = jnp.zeros((16,), jnp.float32)

        def _zrow(i, carry):
            for j in range(D // 16):
                rows_v[i, pl.ds(j * 16, 16)] = zero16
            return carry

        lax.fori_loop(0, cz, _zrow, 0)
        for t in range(nz):
            pltpu.sync_copy(rows_v.at[pl.ds(0, cz)],
                            acc.at[pl.ds(sid * rpt + t * cz, cz)])
        plsc.subcore_barrier()

        def _chunk(c, carry):
            pltpu.sync_copy(w_hbm.at[wid, c], wv)
            pltpu.async_copy(emb_hbm.at[colv.at[c]], rows_v, sem).wait()

            def _scale(e, inner):
                ws = wv[pl.ds(e * 16, 16)]
                for j in range(D // 16):
                    sl = pl.ds(j * 16, 16)
                    rows_v[e, sl] = rows_v[e, sl] * ws
                return inner

            lax.fori_loop(0, _C, _scale, 0)
            pltpu.sync_copy(rows_v, acc.at[rowv.at[c]], add=True)
            return carry

        lax.fori_loop(0, ncw, _chunk, 0)
        plsc.subcore_barrier()

        for t in range(nz):
            sl = pl.ds(sid * rpt + t * cz, cz)
            pltpu.sync_copy(acc.at[sl], part_hbm.at[cid, sl])

    return layer


@functools.lru_cache(maxsize=None)
def _make_combine(Np, D, scale):
    B = _largest_divisor_leq(Np // 8, 128) * 8
    grid = (Np // B,)

    def body(p_ref, a_ref, e_out, a_out):
        s = p_ref[0] + p_ref[1]
        e_out[...] = s
        a_out[...] = (a_ref[...] + s) * scale

    return pl.pallas_call(
        body,
        grid=grid,
        in_specs=[
            pl.BlockSpec((2, B, D), lambda i: (0, i, 0)),
            pl.BlockSpec((B, D), lambda i: (i, 0)),
        ],
        out_specs=[
            pl.BlockSpec((B, D), lambda i: (i, 0)),
            pl.BlockSpec((B, D), lambda i: (i, 0)),
        ],
        out_shape=[
            jax.ShapeDtypeStruct((Np, D), jnp.float32),
            jax.ShapeDtypeStruct((Np, D), jnp.float32),
        ],
    )


def kernel(user_emb_weight, item_emb_weight, edge_index, edge_weight):
    U, D = user_emb_weight.shape
    N = U + item_emb_weight.shape[0]
    E = edge_weight.shape[0]

    all_emb = jnp.concatenate([user_emb_weight, item_emb_weight], axis=0)

    ncw = -(-E // (_NW * _C))
    Ep = _NW * _C * ncw
    row = edge_index[0].astype(jnp.int32)
    col = edge_index[1].astype(jnp.int32)
    w = edge_weight.astype(jnp.float32)
    if Ep > E:
        pad = Ep - E
        row = jnp.concatenate([row, jnp.zeros((pad,), jnp.int32)])
        col = jnp.concatenate([col, jnp.zeros((pad,), jnp.int32)])
        w = jnp.concatenate([w, jnp.zeros((pad,), jnp.float32)])
    col3 = col.reshape(_NW, ncw, _C)
    row3 = row.reshape(_NW, ncw, _C)
    wexp = jnp.repeat(w, 16).reshape(_NW, ncw, _C * 16)

    Np = -(-N // (_NS * 128)) * (_NS * 128)
    emb = all_emb
    if Np > N:
        emb = jnp.concatenate(
            [emb, jnp.zeros((Np - N, D), jnp.float32)], axis=0)

    layer = _make_layer(Np, D, ncw)
    acc = emb
    n_layers = 3
    for l in range(n_layers):
        part = layer(emb, col3, row3, wexp)
        scale = 1.0 / (n_layers + 1) if l == n_layers - 1 else 1.0
        emb, acc = _make_combine(Np, D, scale)(part, acc)

    out = acc[:N]
    return out[:U], out[U:]


# speedup vs baseline: 5.9509x; 1.8666x over previous
import functools

import jax
import jax.numpy as jnp
from jax import lax
from jax.experimental import pallas as pl
from jax.experimental.pallas import tpu as pltpu
from jax.experimental.pallas import tpu_sc as plsc

_NC = 2
_NS = 16
_C = 128
_DH = 64


@functools.lru_cache(maxsize=None)
def _make_layer(Np, ncw):
    rpt = Np // _NS
    nz = rpt // _C
    mesh = plsc.VectorSubcoreMesh(
        core_axis_name="c", subcore_axis_name="s",
        num_cores=_NC, num_subcores=_NS)

    @functools.partial(
        pl.kernel,
        out_type=jax.ShapeDtypeStruct((_NC, Np, _DH), jnp.float32),
        mesh=mesh,
        compiler_params=pltpu.CompilerParams(use_tc_tiling_on_sc=False),
        scratch_types=[
            pltpu.VMEM((ncw, _C), jnp.int32),
            pltpu.VMEM((ncw, _C), jnp.int32),
            pltpu.VMEM((_C, _DH), jnp.float32),
            pltpu.VMEM((_C, _DH), jnp.float32),
            pltpu.VMEM((_C, _DH), jnp.float32),
            pltpu.VMEM((_C, _DH), jnp.float32),
            pltpu.VMEM((_C * 16,), jnp.float32),
            pltpu.VMEM((_C * 16,), jnp.float32),
            pltpu.VMEM_SHARED((Np, _DH), jnp.float32),
            pltpu.SemaphoreType.DMA,
            pltpu.SemaphoreType.DMA,
            pltpu.SemaphoreType.DMA,
            pltpu.SemaphoreType.DMA,
            pltpu.SemaphoreType.DMA,
            pltpu.SemaphoreType.DMA,
        ],
    )
    def layer(tab_hbm, col_hbm, row_hbm, w_hbm, part_hbm,
              colv, rowv, g0, g1, s0, s1, w0, w1, acc,
              gs0, gs1, ss0, ss1, ws0, ws1):
        cid = lax.axis_index("c")
        sid = lax.axis_index("s")
        gbuf = (g0, g1)
        sbuf = (s0, s1)
        wbuf = (w0, w1)
        gsem = (gs0, gs1)
        ssem = (ss0, ss1)
        wsem = (ws0, ws1)

        pltpu.sync_copy(col_hbm.at[cid, sid], colv)
        pltpu.sync_copy(row_hbm.at[sid], rowv)

        zero16 = jnp.zeros((16,), jnp.float32)

        def _zrow(i, carry):
            for j in range(_DH // 16):
                g0[i, pl.ds(j * 16, 16)] = zero16
            return carry

        lax.fori_loop(0, _C, _zrow, 0)
        for t in range(nz):
            pltpu.sync_copy(g0, acc.at[pl.ds(sid * rpt + t * _C, _C)])
        plsc.subcore_barrier()

        def _gather(ch, b):
            return pltpu.make_async_copy(
                tab_hbm.at[colv.at[ch]], gbuf[b], gsem[b])

        def _wload(ch, b):
            return pltpu.make_async_copy(
                w_hbm.at[sid, ch], wbuf[b], wsem[b])

        def _scatter(ch, b):
            return pltpu.make_async_copy(
                sbuf[b], acc.at[rowv.at[ch]], ssem[b])

        for b in range(2):
            _gather(b, b).start()
            _wload(b, b).start()

        def _group(g, carry):
            for b in range(2):
                ch = g * 2 + b
                _gather(ch, b).wait()
                _wload(ch, b).wait()

                @pl.when(g > 0)
                def _():
                    _scatter(ch - 2, b).wait()

                def _scale(e, inner):
                    ws = wbuf[b][pl.ds(e * 16, 16)]
                    for j in range(_DH // 16):
                        sl = pl.ds(j * 16, 16)
                        sbuf[b][e, sl] = gbuf[b][e, sl] * ws
                    return inner

                lax.fori_loop(0, _C, _scale, 0)
                _scatter(ch, b).start(add=True)

                @pl.when(ch + 2 < ncw)
                def _():
                    _gather(ch + 2, b).start()
                    _wload(ch + 2, b).start()
            return carry

        lax.fori_loop(0, ncw // 2, _group, 0)
        for b in range(2):
            _scatter(ncw - 2 + b, b).wait()
        plsc.subcore_barrier()

        for t in range(nz):
            sl = pl.ds(sid * rpt + t * _C, _C)
            pltpu.sync_copy(acc.at[sl], part_hbm.at[cid, sl])

    return layer


@functools.lru_cache(maxsize=None)
def _make_combine(Nr, D, scale):
    B = 1024
    while Nr % B:
        B //= 2
    grid = (Nr // B,)

    def body(p_ref, a_ref, a_out):
        a_out[...] = (a_ref[...] + p_ref[...]) * scale

    return pl.pallas_call(
        body,
        grid=grid,
        in_specs=[
            pl.BlockSpec((B, D), lambda i: (i, 0)),
            pl.BlockSpec((B, D), lambda i: (i, 0)),
        ],
        out_specs=pl.BlockSpec((B, D), lambda i: (i, 0)),
        out_shape=jax.ShapeDtypeStruct((Nr, D), jnp.float32),
    )


def kernel(user_emb_weight, item_emb_weight, edge_index, edge_weight):
    U, D = user_emb_weight.shape
    N = U + item_emb_weight.shape[0]
    E = edge_weight.shape[0]

    all_emb = jnp.concatenate([user_emb_weight, item_emb_weight], axis=0)

    Np = -(-N // (_NS * _C)) * (_NS * _C)
    emb = all_emb
    if Np > N:
        emb = jnp.concatenate(
            [emb, jnp.zeros((Np - N, D), jnp.float32)], axis=0)

    ncw = -(-E // (_NS * _C))
    ncw += ncw % 2
    Ep = _NS * _C * ncw
    row = edge_index[0].astype(jnp.int32)
    col = edge_index[1].astype(jnp.int32)
    w = edge_weight.astype(jnp.float32)
    if Ep > E:
        pad = Ep - E
        row = jnp.concatenate([row, jnp.zeros((pad,), jnp.int32)])
        col = jnp.concatenate([col, jnp.zeros((pad,), jnp.int32)])
        w = jnp.concatenate([w, jnp.zeros((pad,), jnp.float32)])
    col3 = col.reshape(_NS, ncw, _C)
    col4 = jnp.stack([col3, col3 + Np])
    row3 = row.reshape(_NS, ncw, _C)
    wexp = jnp.repeat(w, 16).reshape(_NS, ncw, _C * 16)

    emb2 = jnp.stack([emb[:, :_DH], emb[:, _DH:]])

    layer = _make_layer(Np, ncw)
    acc = emb2.reshape(2 * Np, _DH)
    tab = acc
    n_layers = 3
    for l in range(n_layers):
        part = layer(tab, col4, row3, wexp)
        tab = part.reshape(2 * Np, _DH)
        scale = 1.0 / (n_layers + 1) if l == n_layers - 1 else 1.0
        acc = _make_combine(2 * Np, _DH, scale)(tab, acc)

    out = acc.reshape(2, Np, _DH).transpose(1, 0, 2).reshape(Np, D)[:N]
    return out[:U], out[U:]
